# Initial kernel scaffold; baseline (speedup 1.0000x reference)
import jax
import jax.numpy as jnp
from jax.experimental import pallas as pl


def kernel(x, edge_index, edge_features, global_features, params):
    raise NotImplementedError("write your pallas kernel here")


# speedup vs baseline: 11.3055x; 11.3055x over previous
import functools

import jax
import jax.numpy as jnp
from jax import lax
from jax.experimental import pallas as pl
from jax.experimental.pallas import tpu as pltpu
from jax.experimental.pallas import tpu_sc as plsc

_N = 10000
_E = 320000
_L = 128

_NPAD = 10240
_NW = 32
_ROWS = _NPAD // _NW
_CAPS = 12400
_CAPA = _CAPS + 16
_RCAP = _NW * _CAPS
_ET = _E + _RCAP
_CPW = _ET // _NW // 128

def _mesh():
    return plsc.VectorSubcoreMesh(core_axis_name="c", subcore_axis_name="s")


def _bf16_round(v):
    b = plsc.bitcast(v, jnp.int32)
    r = (b + 0x7FFF + ((b >> 16) & 1)) & (-65536)
    return plsc.bitcast(r, jnp.float32)


def _sc_radius_body(cx_hbm, cy_hbm, src_o, dst_o, dx_o, dy_o, val_o,
                    cx_v, cy_v, cxb_v, cyb_v, sb, db, xb, yb, vb):
    w = lax.axis_index("c") * 16 + lax.axis_index("s")
    pltpu.sync_copy(cx_hbm, cx_v)
    pltpu.sync_copy(cy_hbm, cy_v)
    zi = jnp.zeros((16,), jnp.int32)
    zf = jnp.zeros((16,), jnp.float32)

    def cbody(k, carry):
        off = k * 16
        cxb_v[pl.ds(off, 16)] = _bf16_round(cx_v[pl.ds(off, 16)])
        cyb_v[pl.ds(off, 16)] = _bf16_round(cy_v[pl.ds(off, 16)])
        return carry

    lax.fori_loop(0, _NPAD // 16, cbody, 0)

    def zbody(k, carry):
        off = k * 16
        sb[pl.ds(off, 16)] = zi
        db[pl.ds(off, 16)] = zi
        xb[pl.ds(off, 16)] = zf
        yb[pl.ds(off, 16)] = zf
        vb[pl.ds(off, 16)] = zf
        return carry

    lax.fori_loop(0, _CAPA // 16, zbody, 0)

    iota = lax.iota(jnp.int32, 16)
    ones = jnp.ones((16,), jnp.float32)

    def rbody(r, cnt):
        i = w * _ROWS + r
        isp = jnp.full((16,), i, jnp.int32)
        ib = (i // 16) * 16
        sel = iota == (i - ib)
        cxi_s = jnp.sum(jnp.where(sel, cx_v[pl.ds(ib, 16)], 0.0))
        cyi_s = jnp.sum(jnp.where(sel, cy_v[pl.ds(ib, 16)], 0.0))
        cxbi_s = jnp.sum(jnp.where(sel, cxb_v[pl.ds(ib, 16)], 0.0))
        cybi_s = jnp.sum(jnp.where(sel, cyb_v[pl.ds(ib, 16)], 0.0))
        cxi = jnp.full((16,), cxi_s, jnp.float32)
        cyi = jnp.full((16,), cyi_s, jnp.float32)
        cxbi = jnp.full((16,), cxbi_s, jnp.float32)
        cybi = jnp.full((16,), cybi_s, jnp.float32)
        sqi = cxi * cxi + cyi * cyi

        def jbody(jj, cnt):
            jo = jj * 16
            cxj = cx_v[pl.ds(jo, 16)]
            cyj = cy_v[pl.ds(jo, 16)]
            dot = cxb_v[pl.ds(jo, 16)] * cxbi + cyb_v[pl.ds(jo, 16)] * cybi
            sqj = cxj * cxj + cyj * cyj
            d2 = (sqi + sqj) - 2.0 * dot
            jidx = jo + iota
            msk = (d2 < 0.01) & (jidx != isp)
            c = jnp.minimum(cnt, _CAPS)
            plsc.store_compressed(sb.at[pl.ds(c, 16)], isp, mask=msk)
            plsc.store_compressed(db.at[pl.ds(c, 16)], jidx, mask=msk)
            plsc.store_compressed(xb.at[pl.ds(c, 16)], cxj - cxi, mask=msk)
            plsc.store_compressed(yb.at[pl.ds(c, 16)], cyj - cyi, mask=msk)
            plsc.store_compressed(vb.at[pl.ds(c, 16)], ones, mask=msk)
            return cnt + jnp.sum(msk.astype(jnp.int32))

        return lax.fori_loop(0, _NPAD // 16, jbody, cnt)

    lax.fori_loop(0, _ROWS, rbody, jnp.int32(0))
    pltpu.sync_copy(sb.at[pl.ds(0, _CAPS)], src_o.at[w])
    pltpu.sync_copy(db.at[pl.ds(0, _CAPS)], dst_o.at[w])
    pltpu.sync_copy(xb.at[pl.ds(0, _CAPS)], dx_o.at[w])
    pltpu.sync_copy(yb.at[pl.ds(0, _CAPS)], dy_o.at[w])
    pltpu.sync_copy(vb.at[pl.ds(0, _CAPS)], val_o.at[w])


@functools.cache
def _sc_radius_kernel():
    return pl.kernel(
        _sc_radius_body,
        out_type=[
            jax.ShapeDtypeStruct((_NW, _CAPS), jnp.int32),
            jax.ShapeDtypeStruct((_NW, _CAPS), jnp.int32),
            jax.ShapeDtypeStruct((_NW, _CAPS), jnp.float32),
            jax.ShapeDtypeStruct((_NW, _CAPS), jnp.float32),
            jax.ShapeDtypeStruct((_NW, _CAPS), jnp.float32),
        ],
        mesh=_mesh(),
        compiler_params=pltpu.CompilerParams(needs_layout_passes=False, use_tc_tiling_on_sc=False),
        scratch_types=[
            pltpu.VMEM((_NPAD,), jnp.float32),
            pltpu.VMEM((_NPAD,), jnp.float32),
            pltpu.VMEM((_NPAD,), jnp.float32),
            pltpu.VMEM((_NPAD,), jnp.float32),
            pltpu.VMEM((_CAPA,), jnp.int32),
            pltpu.VMEM((_CAPA,), jnp.int32),
            pltpu.VMEM((_CAPA,), jnp.float32),
            pltpu.VMEM((_CAPA,), jnp.float32),
            pltpu.VMEM((_CAPA,), jnp.float32),
        ],
    )


def _sc_radius(cxp, cyp):
    return _sc_radius_kernel()(cxp, cyp)


def _sc_gather_body(u_hbm, idx_hbm, out_hbm, idx_v, rows_v, sem):
    w = lax.axis_index("c") * 16 + lax.axis_index("s")
    base = w * (_CPW * 128)

    def body(t, carry):
        off = base + t * 128
        pltpu.sync_copy(idx_hbm.at[pl.ds(off, 128)], idx_v)
        pltpu.async_copy(u_hbm.at[idx_v], rows_v, sem).wait()
        pltpu.sync_copy(rows_v, out_hbm.at[pl.ds(off, 128)])
        return carry

    lax.fori_loop(0, _CPW, body, 0)


@functools.cache
def _sc_gather_kernel():
    return pl.kernel(
        _sc_gather_body,
        out_type=jax.ShapeDtypeStruct((_ET, _L), jnp.float32),
        mesh=_mesh(),
        compiler_params=pltpu.CompilerParams(needs_layout_passes=False, use_tc_tiling_on_sc=False),
        scratch_types=[
            pltpu.VMEM((128,), jnp.int32),
            pltpu.VMEM((128, _L), jnp.float32),
            pltpu.SemaphoreType.DMA,
        ],
    )


def _sc_gather(u, idx):
    return _sc_gather_kernel()(u, idx)


def _sc_scatter_body(m_hbm, tgt_hbm, zeros_hbm, out_hbm, idx_v, mb, acc):
    c = lax.axis_index("c")
    s = lax.axis_index("s")
    w = c * 16 + s

    def zbody(q, carry):
        r0 = s * 625 + q * 125
        pltpu.sync_copy(zeros_hbm, acc.at[pl.ds(r0, 125)])
        return carry

    lax.fori_loop(0, 5, zbody, 0)
    plsc.subcore_barrier()

    base = w * (_CPW * 128)

    def body(t, carry):
        off = base + t * 128
        pltpu.sync_copy(tgt_hbm.at[pl.ds(off, 128)], idx_v.at[0])
        pltpu.sync_copy(m_hbm.at[pl.ds(off, 128)], mb)
        pltpu.sync_copy(mb, acc.at[idx_v.at[0]], add=True)
        return carry

    lax.fori_loop(0, _CPW, body, 0)
    plsc.subcore_barrier()

    def obody(q, carry):
        r0 = s * 625 + q * 125
        pltpu.sync_copy(acc.at[pl.ds(r0, 125)], out_hbm.at[c, pl.ds(r0, 125)])
        return carry

    lax.fori_loop(0, 5, obody, 0)


@functools.cache
def _sc_scatter_kernel():
    return pl.kernel(
        _sc_scatter_body,
        out_type=jax.ShapeDtypeStruct((2, _N, _L), jnp.float32),
        mesh=_mesh(),
        compiler_params=pltpu.CompilerParams(needs_layout_passes=False, use_tc_tiling_on_sc=False),
        scratch_types=[
            pltpu.VMEM((2, 128), jnp.int32),
            pltpu.VMEM((128, _L), jnp.float32),
            pltpu.VMEM_SHARED((_N, _L), jnp.float32),
        ],
    )


def _sc_scatter(m, tgt, zeros_chunk):
    return _sc_scatter_kernel()(m, tgt, zeros_chunk)


_BN = 1000
_BE = 1024


def _tc_node(xp, gfp, wn1, bn1, wn2, bn2, wg1, bg1, wg2, bg2):
    def body(x_ref, gf_ref, wn1r, bn1r, wn2r, bn2r, wg1r, bg1r, wg2r, bg2r, o_ref):
        h = jnp.maximum(x_ref[...] @ wn1r[...] + bn1r[...], 0.0)
        h = jnp.maximum(h @ wn2r[...] + bn2r[...], 0.0)
        gg = jnp.maximum(gf_ref[...] @ wg1r[...] + bg1r[...], 0.0)
        gg = gg @ wg2r[...] + bg2r[...]
        o_ref[...] = h + gg[0:1, :]

    full = lambda shape: pl.BlockSpec(shape, lambda i: (0, 0))
    return pl.pallas_call(
        body,
        grid=(_N // _BN,),
        in_specs=[
            pl.BlockSpec((_BN, 128), lambda i: (i, 0)),
            full((8, 128)), full((128, 128)), full((1, 128)),
            full((128, 128)), full((1, 128)),
            full((128, 128)), full((1, 128)),
            full((128, 128)), full((1, 128)),
        ],
        out_specs=pl.BlockSpec((_BN, _L), lambda i: (i, 0)),
        out_shape=jax.ShapeDtypeStruct((_N, _L), jnp.float32),
    )(xp, gfp, wn1, bn1, wn2, bn2, wg1, bg1, wg2, bg2)


def _tc_edge_enc(feat, w1, b1, w2, b2):
    def body(f_ref, w1r, b1r, w2r, b2r, o_ref):
        h = jnp.maximum(f_ref[...] @ w1r[...] + b1r[...], 0.0)
        o_ref[...] = jnp.maximum(h @ w2r[...] + b2r[...], 0.0)

    full = lambda shape: pl.BlockSpec(shape, lambda i: (0, 0))
    return pl.pallas_call(
        body,
        grid=(_ET // _BE,),
        in_specs=[
            pl.BlockSpec((_BE, 8), lambda i: (i, 0)),
            full((8, 128)), full((1, 128)),
            full((128, 128)), full((1, 128)),
        ],
        out_specs=pl.BlockSpec((_BE, _L), lambda i: (i, 0)),
        out_shape=jax.ShapeDtypeStruct((_ET, _L), jnp.float32),
    )(feat, w1, b1, w2, b2)


def _tc_lin(x, w, b):
    def body(x_ref, w_ref, b_ref, o_ref):
        o_ref[...] = x_ref[...] @ w_ref[...] + b_ref[...]

    full = lambda shape: pl.BlockSpec(shape, lambda i: (0, 0))
    return pl.pallas_call(
        body,
        grid=(_N // _BN,),
        in_specs=[
            pl.BlockSpec((_BN, 128), lambda i: (i, 0)),
            full((128, 128)), full((1, 128)),
        ],
        out_specs=pl.BlockSpec((_BN, 128), lambda i: (i, 0)),
        out_shape=jax.ShapeDtypeStruct((_N, 128), jnp.float32),
    )(x, w, b)


def _tc_msg(g, e, valid, w1b, w2, b2):
    def body(g_ref, e_ref, v_ref, w1r, w2r, b2r, o_ref):
        m1 = jnp.maximum(g_ref[...] + e_ref[...] @ w1r[...], 0.0)
        m = jnp.maximum(m1 @ w2r[...] + b2r[...], 0.0)
        o_ref[...] = jnp.where(v_ref[...] > 0.0, m, 0.0)

    full = lambda shape: pl.BlockSpec(shape, lambda i: (0, 0))
    return pl.pallas_call(
        body,
        grid=(_ET // _BE,),
        in_specs=[
            pl.BlockSpec((_BE, _L), lambda i: (i, 0)),
            pl.BlockSpec((_BE, _L), lambda i: (i, 0)),
            pl.BlockSpec((_BE, 1), lambda i: (i, 0)),
            full((128, 128)), full((128, 128)), full((1, 128)),
        ],
        out_specs=pl.BlockSpec((_BE, _L), lambda i: (i, 0)),
        out_shape=jax.ShapeDtypeStruct((_ET, _L), jnp.float32),
    )(g, e, valid, w1b, w2, b2)


def _tc_upd(x_enc, p0, p1, wua, wub, b1, w2, b2):
    def body(x_ref, p0_ref, p1_ref, wuar, wubr, b1r, w2r, b2r, o_ref):
        mv = p0_ref[...] + p1_ref[...]
        t = jnp.maximum(x_ref[...] @ wuar[...] + mv @ wubr[...] + b1r[...], 0.0)
        t = jnp.maximum(t @ w2r[...] + b2r[...], 0.0)
        o_ref[...] = jnp.maximum(x_ref[...] + t, 0.0)

    full = lambda shape: pl.BlockSpec(shape, lambda i: (0, 0))
    return pl.pallas_call(
        body,
        grid=(_N // _BN,),
        in_specs=[
            pl.BlockSpec((_BN, _L), lambda i: (i, 0)),
            pl.BlockSpec((_BN, _L), lambda i: (i, 0)),
            pl.BlockSpec((_BN, _L), lambda i: (i, 0)),
            full((128, 128)), full((128, 128)), full((1, 128)),
            full((128, 128)), full((1, 128)),
        ],
        out_specs=pl.BlockSpec((_BN, _L), lambda i: (i, 0)),
        out_shape=jax.ShapeDtypeStruct((_N, _L), jnp.float32),
    )(x_enc, p0, p1, wua, wub, b1, w2, b2)


def _tc_head(x_enc, wr, wz, wn, br, bz, bn, hr, hz, hn, wd1, bd1, wd2, bd2, wd3, bd3):
    def body(x_ref, wrr, wzr, wnr, brr, bzr, bnr, hrr, hzr, hnr,
             wd1r, bd1r, wd2r, bd2r, wd3r, bd3r, o_ref, mem_ref):
        x = x_ref[...]
        r = jax.nn.sigmoid(x @ wrr[...] + brr[...] + hrr[...])
        z = jax.nn.sigmoid(x @ wzr[...] + bzr[...] + hzr[...])
        nn_ = jnp.tanh(x @ wnr[...] + bnr[...] + r * hnr[...])
        mem = (1.0 - z) * nn_
        h = jnp.maximum(mem @ wd1r[...] + bd1r[...], 0.0)
        h = jnp.maximum(h @ wd2r[...] + bd2r[...], 0.0)
        o_ref[...] = h @ wd3r[...] + bd3r[...]
        mem_ref[...] = mem

    full = lambda shape: pl.BlockSpec(shape, lambda i: (0, 0))
    return pl.pallas_call(
        body,
        grid=(_N // _BN,),
        in_specs=[pl.BlockSpec((_BN, _L), lambda i: (i, 0))]
        + [full((128, 128)), full((128, 128)), full((128, 128))]
        + [full((1, 128))] * 6
        + [full((128, 128)), full((1, 128))] * 3,
        out_specs=[
            pl.BlockSpec((_BN, 128), lambda i: (i, 0)),
            pl.BlockSpec((_BN, _L), lambda i: (i, 0)),
        ],
        out_shape=[
            jax.ShapeDtypeStruct((_N, 128), jnp.float32),
            jax.ShapeDtypeStruct((_N, _L), jnp.float32),
        ],
    )(x_enc, wr, wz, wn, br, bz, bn, hr, hz, hn, wd1, bd1, wd2, bd2, wd3, bd3)


def _row(b):
    return b.reshape(1, -1)


def kernel(x, edge_index, edge_features, global_features, params):
    f32 = jnp.float32
    coords = x[:, :2]
    pad = 100.0 + 4.0 * jnp.arange(_NPAD - _N, dtype=f32)
    cxp = jnp.concatenate([coords[:, 0], pad])
    cyp = jnp.concatenate([coords[:, 1], jnp.zeros((_NPAD - _N,), f32)])

    src32, dst32, dx32, dy32, val32 = _sc_radius(cxp, cyp)
    src_r = src32.reshape(-1)
    dst_r = dst32.reshape(-1)

    src_all = jnp.concatenate([edge_index[0].astype(jnp.int32), src_r])
    tgt_all = jnp.concatenate([edge_index[1].astype(jnp.int32), dst_r])
    valid = jnp.concatenate([jnp.ones((_E, 1), f32), val32.reshape(-1, 1)])

    feat = jnp.concatenate([
        jnp.pad(edge_features, ((0, 0), (0, 6))),
        jnp.concatenate([dx32.reshape(-1, 1), dy32.reshape(-1, 1),
                         jnp.zeros((_RCAP, 6), f32)], axis=1),
    ], axis=0)

    p = params
    xp = jnp.pad(x, ((0, 0), (0, 125)))
    gfp = jnp.pad(global_features.reshape(1, 3), ((0, 7), (0, 125)))

    wn1 = jnp.pad(p["node"]["l1"]["W"].T, ((0, 125), (0, 0)))
    wg1 = jnp.pad(p["glob"]["l1"]["W"].T, ((0, 125), (0, 0)))
    x_enc = _tc_node(xp, gfp,
                     wn1, _row(p["node"]["l1"]["b"]),
                     p["node"]["l2"]["W"].T, _row(p["node"]["l2"]["b"]),
                     wg1, _row(p["glob"]["l1"]["b"]),
                     p["glob"]["l2"]["W"].T, _row(p["glob"]["l2"]["b"]))

    we1 = jnp.pad(p["edge"]["l1"]["W"].T, ((0, 6), (0, 0)))
    e_all = _tc_edge_enc(feat, we1, _row(p["edge"]["l1"]["b"]),
                         p["edge"]["l2"]["W"].T, _row(p["edge"]["l2"]["b"]))

    zeros_chunk = jnp.zeros((125, _L), f32)
    for pm, pu in zip(p["msg"], p["upd"]):
        w1 = pm["l1"]["W"]
        u = _tc_lin(x_enc, w1[:, :_L].T, _row(pm["l1"]["b"]))
        gth = _sc_gather(u, src_all)
        m = _tc_msg(gth, e_all, valid, w1[:, _L:].T,
                    pm["l2"]["W"].T, _row(pm["l2"]["b"]))
        parts = _sc_scatter(m, tgt_all, zeros_chunk)
        wu = pu["l1"]["W"]
        x_enc = _tc_upd(x_enc, parts[0], parts[1],
                        wu[:, :_L].T, wu[:, _L:].T, _row(pu["l1"]["b"]),
                        pu["l2"]["W"].T, _row(pu["l2"]["b"]))

    gru = p["gru"]
    wih = gru["Wih"]
    bih = gru["bih"]
    bhh = gru["bhh"]
    d = p["dec"]
    wd3 = jnp.pad(d["l3"]["W"].T, ((0, 0), (0, 125)))
    bd3 = jnp.pad(_row(d["l3"]["b"]), ((0, 0), (0, 125)))
    out_pad, memory = _tc_head(
        x_enc,
        wih[:_L].T, wih[_L:2 * _L].T, wih[2 * _L:].T,
        _row(bih[:_L]), _row(bih[_L:2 * _L]), _row(bih[2 * _L:]),
        _row(bhh[:_L]), _row(bhh[_L:2 * _L]), _row(bhh[2 * _L:]),
        d["l1"]["W"].T, _row(d["l1"]["b"]),
        d["l2"]["W"].T, _row(d["l2"]["b"]),
        wd3, bd3)
    return out_pad[:, :3], memory


# speedup vs baseline: 11.3847x; 1.0070x over previous
import functools

import jax
import jax.numpy as jnp
from jax import lax
from jax.experimental import pallas as pl
from jax.experimental.pallas import tpu as pltpu
from jax.experimental.pallas import tpu_sc as plsc

_N = 10000
_E = 320000
_L = 128

_NPAD = 10240
_NW = 32
_ROWS = _NPAD // _NW
_CAPS = 12400
_CAPA = _CAPS + 16
_RCAP = _NW * _CAPS
_ET = _E + _RCAP
_CPW = _ET // _NW // 128

def _mesh():
    return plsc.VectorSubcoreMesh(core_axis_name="c", subcore_axis_name="s")


def _bf16_round(v):
    b = plsc.bitcast(v, jnp.int32)
    r = (b + 0x7FFF + ((b >> 16) & 1)) & (-65536)
    return plsc.bitcast(r, jnp.float32)


def _sc_radius_body(cx_hbm, cy_hbm, src_o, dst_o, dx_o, dy_o, val_o,
                    cx_v, cy_v, cxb_v, cyb_v, sb, db, xb, yb, vb):
    w = lax.axis_index("c") * 16 + lax.axis_index("s")
    pltpu.sync_copy(cx_hbm, cx_v)
    pltpu.sync_copy(cy_hbm, cy_v)
    zi = jnp.zeros((16,), jnp.int32)
    zf = jnp.zeros((16,), jnp.float32)

    def cbody(k, carry):
        off = k * 16
        cxb_v[pl.ds(off, 16)] = _bf16_round(cx_v[pl.ds(off, 16)])
        cyb_v[pl.ds(off, 16)] = _bf16_round(cy_v[pl.ds(off, 16)])
        return carry

    lax.fori_loop(0, _NPAD // 16, cbody, 0)

    def zbody(k, carry):
        off = k * 16
        sb[pl.ds(off, 16)] = zi
        db[pl.ds(off, 16)] = zi
        xb[pl.ds(off, 16)] = zf
        yb[pl.ds(off, 16)] = zf
        vb[pl.ds(off, 16)] = zf
        return carry

    lax.fori_loop(0, _CAPA // 16, zbody, 0)

    iota = lax.iota(jnp.int32, 16)
    ones = jnp.ones((16,), jnp.float32)

    def rbody(r, cnt):
        i = w * _ROWS + r
        isp = jnp.full((16,), i, jnp.int32)
        ib = (i // 16) * 16
        sel = iota == (i - ib)
        cxi_s = jnp.sum(jnp.where(sel, cx_v[pl.ds(ib, 16)], 0.0))
        cyi_s = jnp.sum(jnp.where(sel, cy_v[pl.ds(ib, 16)], 0.0))
        cxbi_s = jnp.sum(jnp.where(sel, cxb_v[pl.ds(ib, 16)], 0.0))
        cybi_s = jnp.sum(jnp.where(sel, cyb_v[pl.ds(ib, 16)], 0.0))
        cxi = jnp.full((16,), cxi_s, jnp.float32)
        cyi = jnp.full((16,), cyi_s, jnp.float32)
        cxbi = jnp.full((16,), cxbi_s, jnp.float32)
        cybi = jnp.full((16,), cybi_s, jnp.float32)
        sqi = cxi * cxi + cyi * cyi

        def jbody(jj, cnt):
            jo = jj * 16
            cxj = cx_v[pl.ds(jo, 16)]
            cyj = cy_v[pl.ds(jo, 16)]
            dot = cxb_v[pl.ds(jo, 16)] * cxbi + cyb_v[pl.ds(jo, 16)] * cybi
            sqj = cxj * cxj + cyj * cyj
            d2 = (sqi + sqj) - 2.0 * dot
            jidx = jo + iota
            msk = (d2 < 0.01) & (jidx != isp)
            c = jnp.minimum(cnt, _CAPS)
            plsc.store_compressed(sb.at[pl.ds(c, 16)], isp, mask=msk)
            plsc.store_compressed(db.at[pl.ds(c, 16)], jidx, mask=msk)
            plsc.store_compressed(xb.at[pl.ds(c, 16)], cxj - cxi, mask=msk)
            plsc.store_compressed(yb.at[pl.ds(c, 16)], cyj - cyi, mask=msk)
            plsc.store_compressed(vb.at[pl.ds(c, 16)], ones, mask=msk)
            return cnt + jnp.sum(msk.astype(jnp.int32))

        return lax.fori_loop(0, _NPAD // 16, jbody, cnt)

    lax.fori_loop(0, _ROWS, rbody, jnp.int32(0))
    pltpu.sync_copy(sb.at[pl.ds(0, _CAPS)], src_o.at[w])
    pltpu.sync_copy(db.at[pl.ds(0, _CAPS)], dst_o.at[w])
    pltpu.sync_copy(xb.at[pl.ds(0, _CAPS)], dx_o.at[w])
    pltpu.sync_copy(yb.at[pl.ds(0, _CAPS)], dy_o.at[w])
    pltpu.sync_copy(vb.at[pl.ds(0, _CAPS)], val_o.at[w])


@functools.cache
def _sc_radius_kernel():
    return pl.kernel(
        _sc_radius_body,
        out_type=[
            jax.ShapeDtypeStruct((_NW, _CAPS), jnp.int32),
            jax.ShapeDtypeStruct((_NW, _CAPS), jnp.int32),
            jax.ShapeDtypeStruct((_NW, _CAPS), jnp.float32),
            jax.ShapeDtypeStruct((_NW, _CAPS), jnp.float32),
            jax.ShapeDtypeStruct((_NW, _CAPS), jnp.float32),
        ],
        mesh=_mesh(),
        compiler_params=pltpu.CompilerParams(needs_layout_passes=False, use_tc_tiling_on_sc=False),
        scratch_types=[
            pltpu.VMEM((_NPAD,), jnp.float32),
            pltpu.VMEM((_NPAD,), jnp.float32),
            pltpu.VMEM((_NPAD,), jnp.float32),
            pltpu.VMEM((_NPAD,), jnp.float32),
            pltpu.VMEM((_CAPA,), jnp.int32),
            pltpu.VMEM((_CAPA,), jnp.int32),
            pltpu.VMEM((_CAPA,), jnp.float32),
            pltpu.VMEM((_CAPA,), jnp.float32),
            pltpu.VMEM((_CAPA,), jnp.float32),
        ],
    )


def _sc_radius(cxp, cyp):
    return _sc_radius_kernel()(cxp, cyp)


_NBUF = 5
_TPW = _CPW // _NBUF


def _sc_gather_body(u_hbm, idx_hbm, out_hbm, idx_all, r0, r1, r2, r3, r4,
                    g0, g1, g2, g3, g4, w0, w1, w2, w3, w4):
    w = lax.axis_index("c") * 16 + lax.axis_index("s")
    base = w * (_CPW * 128)
    rows = [r0, r1, r2, r3, r4]
    gs = [g0, g1, g2, g3, g4]
    ws = [w0, w1, w2, w3, w4]

    pltpu.sync_copy(idx_hbm.at[pl.ds(base, _CPW * 128)], idx_all)
    for b in range(_NBUF):
        pltpu.async_copy(u_hbm.at[idx_all.at[pl.ds(b * 128, 128)]], rows[b], gs[b])

    def mbody(tb, carry):
        for b in range(_NBUF):
            t = tb * _NBUF + b
            off = base + t * 128
            pltpu.make_async_copy(
                u_hbm.at[idx_all.at[pl.ds(0, 128)]], rows[b], gs[b]).wait()
            pltpu.async_copy(rows[b], out_hbm.at[pl.ds(off, 128)], ws[b])

            @pl.when(tb < _TPW - 1)
            def _():
                pltpu.make_async_copy(
                    rows[b], out_hbm.at[pl.ds(base, 128)], ws[b]).wait()
                pltpu.async_copy(
                    u_hbm.at[idx_all.at[pl.ds((t + _NBUF) * 128, 128)]],
                    rows[b], gs[b])
        return carry

    lax.fori_loop(0, _TPW, mbody, 0)
    for b in range(_NBUF):
        pltpu.make_async_copy(rows[b], out_hbm.at[pl.ds(base, 128)], ws[b]).wait()


@functools.cache
def _sc_gather_kernel():
    return pl.kernel(
        _sc_gather_body,
        out_type=jax.ShapeDtypeStruct((_ET, _L), jnp.float32),
        mesh=_mesh(),
        compiler_params=pltpu.CompilerParams(needs_layout_passes=False, use_tc_tiling_on_sc=False),
        scratch_types=[
            pltpu.VMEM((_CPW * 128,), jnp.int32),
        ]
        + [pltpu.VMEM((128, _L), jnp.float32)] * _NBUF
        + [pltpu.SemaphoreType.DMA] * (2 * _NBUF),
    )


def _sc_gather(u, idx):
    return _sc_gather_kernel()(u, idx)


def _sc_scatter_body(m_hbm, tgt_hbm, zeros_hbm, out_hbm, idx_v, mb, acc):
    c = lax.axis_index("c")
    s = lax.axis_index("s")
    w = c * 16 + s

    def zbody(q, carry):
        r0 = s * 625 + q * 125
        pltpu.sync_copy(zeros_hbm, acc.at[pl.ds(r0, 125)])
        return carry

    lax.fori_loop(0, 5, zbody, 0)
    plsc.subcore_barrier()

    base = w * (_CPW * 128)

    def body(t, carry):
        off = base + t * 128
        pltpu.sync_copy(tgt_hbm.at[pl.ds(off, 128)], idx_v.at[0])
        pltpu.sync_copy(m_hbm.at[pl.ds(off, 128)], mb)
        pltpu.sync_copy(mb, acc.at[idx_v.at[0]], add=True)
        return carry

    lax.fori_loop(0, _CPW, body, 0)
    plsc.subcore_barrier()

    def obody(q, carry):
        r0 = s * 625 + q * 125
        pltpu.sync_copy(acc.at[pl.ds(r0, 125)], out_hbm.at[c, pl.ds(r0, 125)])
        return carry

    lax.fori_loop(0, 5, obody, 0)


@functools.cache
def _sc_scatter_kernel():
    return pl.kernel(
        _sc_scatter_body,
        out_type=jax.ShapeDtypeStruct((2, _N, _L), jnp.float32),
        mesh=_mesh(),
        compiler_params=pltpu.CompilerParams(needs_layout_passes=False, use_tc_tiling_on_sc=False),
        scratch_types=[
            pltpu.VMEM((2, 128), jnp.int32),
            pltpu.VMEM((128, _L), jnp.float32),
            pltpu.VMEM_SHARED((_N, _L), jnp.float32),
        ],
    )


def _sc_scatter(m, tgt, zeros_chunk):
    return _sc_scatter_kernel()(m, tgt, zeros_chunk)


_BN = 1000
_BE = 1024


def _tc_node(xp, gfp, wn1, bn1, wn2, bn2, wg1, bg1, wg2, bg2):
    def body(x_ref, gf_ref, wn1r, bn1r, wn2r, bn2r, wg1r, bg1r, wg2r, bg2r, o_ref):
        h = jnp.maximum(x_ref[...] @ wn1r[...] + bn1r[...], 0.0)
        h = jnp.maximum(h @ wn2r[...] + bn2r[...], 0.0)
        gg = jnp.maximum(gf_ref[...] @ wg1r[...] + bg1r[...], 0.0)
        gg = gg @ wg2r[...] + bg2r[...]
        o_ref[...] = h + gg[0:1, :]

    full = lambda shape: pl.BlockSpec(shape, lambda i: (0, 0))
    return pl.pallas_call(
        body,
        grid=(_N // _BN,),
        in_specs=[
            pl.BlockSpec((_BN, 128), lambda i: (i, 0)),
            full((8, 128)), full((128, 128)), full((1, 128)),
            full((128, 128)), full((1, 128)),
            full((128, 128)), full((1, 128)),
            full((128, 128)), full((1, 128)),
        ],
        out_specs=pl.BlockSpec((_BN, _L), lambda i: (i, 0)),
        out_shape=jax.ShapeDtypeStruct((_N, _L), jnp.float32),
    )(xp, gfp, wn1, bn1, wn2, bn2, wg1, bg1, wg2, bg2)


def _tc_edge_enc(feat, w1, b1, w2, b2):
    def body(f_ref, w1r, b1r, w2r, b2r, o_ref):
        h = jnp.maximum(f_ref[...] @ w1r[...] + b1r[...], 0.0)
        o_ref[...] = jnp.maximum(h @ w2r[...] + b2r[...], 0.0)

    full = lambda shape: pl.BlockSpec(shape, lambda i: (0, 0))
    return pl.pallas_call(
        body,
        grid=(_ET // _BE,),
        in_specs=[
            pl.BlockSpec((_BE, 8), lambda i: (i, 0)),
            full((8, 128)), full((1, 128)),
            full((128, 128)), full((1, 128)),
        ],
        out_specs=pl.BlockSpec((_BE, _L), lambda i: (i, 0)),
        out_shape=jax.ShapeDtypeStruct((_ET, _L), jnp.float32),
    )(feat, w1, b1, w2, b2)


def _tc_lin(x, w, b):
    def body(x_ref, w_ref, b_ref, o_ref):
        o_ref[...] = x_ref[...] @ w_ref[...] + b_ref[...]

    full = lambda shape: pl.BlockSpec(shape, lambda i: (0, 0))
    return pl.pallas_call(
        body,
        grid=(_N // _BN,),
        in_specs=[
            pl.BlockSpec((_BN, 128), lambda i: (i, 0)),
            full((128, 128)), full((1, 128)),
        ],
        out_specs=pl.BlockSpec((_BN, 128), lambda i: (i, 0)),
        out_shape=jax.ShapeDtypeStruct((_N, 128), jnp.float32),
    )(x, w, b)


def _tc_msg(g, e, valid, w1b, w2, b2):
    def body(g_ref, e_ref, v_ref, w1r, w2r, b2r, o_ref):
        m1 = jnp.maximum(g_ref[...] + e_ref[...] @ w1r[...], 0.0)
        m = jnp.maximum(m1 @ w2r[...] + b2r[...], 0.0)
        o_ref[...] = jnp.where(v_ref[...] > 0.0, m, 0.0)

    full = lambda shape: pl.BlockSpec(shape, lambda i: (0, 0))
    return pl.pallas_call(
        body,
        grid=(_ET // _BE,),
        in_specs=[
            pl.BlockSpec((_BE, _L), lambda i: (i, 0)),
            pl.BlockSpec((_BE, _L), lambda i: (i, 0)),
            pl.BlockSpec((_BE, 1), lambda i: (i, 0)),
            full((128, 128)), full((128, 128)), full((1, 128)),
        ],
        out_specs=pl.BlockSpec((_BE, _L), lambda i: (i, 0)),
        out_shape=jax.ShapeDtypeStruct((_ET, _L), jnp.float32),
    )(g, e, valid, w1b, w2, b2)


def _tc_upd(x_enc, p0, p1, wua, wub, b1, w2, b2):
    def body(x_ref, p0_ref, p1_ref, wuar, wubr, b1r, w2r, b2r, o_ref):
        mv = p0_ref[...] + p1_ref[...]
        t = jnp.maximum(x_ref[...] @ wuar[...] + mv @ wubr[...] + b1r[...], 0.0)
        t = jnp.maximum(t @ w2r[...] + b2r[...], 0.0)
        o_ref[...] = jnp.maximum(x_ref[...] + t, 0.0)

    full = lambda shape: pl.BlockSpec(shape, lambda i: (0, 0))
    return pl.pallas_call(
        body,
        grid=(_N // _BN,),
        in_specs=[
            pl.BlockSpec((_BN, _L), lambda i: (i, 0)),
            pl.BlockSpec((_BN, _L), lambda i: (i, 0)),
            pl.BlockSpec((_BN, _L), lambda i: (i, 0)),
            full((128, 128)), full((128, 128)), full((1, 128)),
            full((128, 128)), full((1, 128)),
        ],
        out_specs=pl.BlockSpec((_BN, _L), lambda i: (i, 0)),
        out_shape=jax.ShapeDtypeStruct((_N, _L), jnp.float32),
    )(x_enc, p0, p1, wua, wub, b1, w2, b2)


def _tc_head(x_enc, wr, wz, wn, br, bz, bn, hr, hz, hn, wd1, bd1, wd2, bd2, wd3, bd3):
    def body(x_ref, wrr, wzr, wnr, brr, bzr, bnr, hrr, hzr, hnr,
             wd1r, bd1r, wd2r, bd2r, wd3r, bd3r, o_ref, mem_ref):
        x = x_ref[...]
        r = jax.nn.sigmoid(x @ wrr[...] + brr[...] + hrr[...])
        z = jax.nn.sigmoid(x @ wzr[...] + bzr[...] + hzr[...])
        nn_ = jnp.tanh(x @ wnr[...] + bnr[...] + r * hnr[...])
        mem = (1.0 - z) * nn_
        h = jnp.maximum(mem @ wd1r[...] + bd1r[...], 0.0)
        h = jnp.maximum(h @ wd2r[...] + bd2r[...], 0.0)
        o_ref[...] = h @ wd3r[...] + bd3r[...]
        mem_ref[...] = mem

    full = lambda shape: pl.BlockSpec(shape, lambda i: (0, 0))
    return pl.pallas_call(
        body,
        grid=(_N // _BN,),
        in_specs=[pl.BlockSpec((_BN, _L), lambda i: (i, 0))]
        + [full((128, 128)), full((128, 128)), full((128, 128))]
        + [full((1, 128))] * 6
        + [full((128, 128)), full((1, 128))] * 3,
        out_specs=[
            pl.BlockSpec((_BN, 128), lambda i: (i, 0)),
            pl.BlockSpec((_BN, _L), lambda i: (i, 0)),
        ],
        out_shape=[
            jax.ShapeDtypeStruct((_N, 128), jnp.float32),
            jax.ShapeDtypeStruct((_N, _L), jnp.float32),
        ],
    )(x_enc, wr, wz, wn, br, bz, bn, hr, hz, hn, wd1, bd1, wd2, bd2, wd3, bd3)


def _row(b):
    return b.reshape(1, -1)


def kernel(x, edge_index, edge_features, global_features, params):
    f32 = jnp.float32
    coords = x[:, :2]
    pad = 100.0 + 4.0 * jnp.arange(_NPAD - _N, dtype=f32)
    cxp = jnp.concatenate([coords[:, 0], pad])
    cyp = jnp.concatenate([coords[:, 1], jnp.zeros((_NPAD - _N,), f32)])

    src32, dst32, dx32, dy32, val32 = _sc_radius(cxp, cyp)
    src_r = src32.reshape(-1)
    dst_r = dst32.reshape(-1)

    src_all = jnp.concatenate([edge_index[0].astype(jnp.int32), src_r])
    tgt_all = jnp.concatenate([edge_index[1].astype(jnp.int32), dst_r])
    valid = jnp.concatenate([jnp.ones((_E, 1), f32), val32.reshape(-1, 1)])

    feat = jnp.concatenate([
        jnp.pad(edge_features, ((0, 0), (0, 6))),
        jnp.concatenate([dx32.reshape(-1, 1), dy32.reshape(-1, 1),
                         jnp.zeros((_RCAP, 6), f32)], axis=1),
    ], axis=0)

    p = params
    xp = jnp.pad(x, ((0, 0), (0, 125)))
    gfp = jnp.pad(global_features.reshape(1, 3), ((0, 7), (0, 125)))

    wn1 = jnp.pad(p["node"]["l1"]["W"].T, ((0, 125), (0, 0)))
    wg1 = jnp.pad(p["glob"]["l1"]["W"].T, ((0, 125), (0, 0)))
    x_enc = _tc_node(xp, gfp,
                     wn1, _row(p["node"]["l1"]["b"]),
                     p["node"]["l2"]["W"].T, _row(p["node"]["l2"]["b"]),
                     wg1, _row(p["glob"]["l1"]["b"]),
                     p["glob"]["l2"]["W"].T, _row(p["glob"]["l2"]["b"]))

    we1 = jnp.pad(p["edge"]["l1"]["W"].T, ((0, 6), (0, 0)))
    e_all = _tc_edge_enc(feat, we1, _row(p["edge"]["l1"]["b"]),
                         p["edge"]["l2"]["W"].T, _row(p["edge"]["l2"]["b"]))

    zeros_chunk = jnp.zeros((125, _L), f32)
    for pm, pu in zip(p["msg"], p["upd"]):
        w1 = pm["l1"]["W"]
        u = _tc_lin(x_enc, w1[:, :_L].T, _row(pm["l1"]["b"]))
        gth = _sc_gather(u, src_all)
        m = _tc_msg(gth, e_all, valid, w1[:, _L:].T,
                    pm["l2"]["W"].T, _row(pm["l2"]["b"]))
        parts = _sc_scatter(m, tgt_all, zeros_chunk)
        wu = pu["l1"]["W"]
        x_enc = _tc_upd(x_enc, parts[0], parts[1],
                        wu[:, :_L].T, wu[:, _L:].T, _row(pu["l1"]["b"]),
                        pu["l2"]["W"].T, _row(pu["l2"]["b"]))

    gru = p["gru"]
    wih = gru["Wih"]
    bih = gru["bih"]
    bhh = gru["bhh"]
    d = p["dec"]
    wd3 = jnp.pad(d["l3"]["W"].T, ((0, 0), (0, 125)))
    bd3 = jnp.pad(_row(d["l3"]["b"]), ((0, 0), (0, 125)))
    out_pad, memory = _tc_head(
        x_enc,
        wih[:_L].T, wih[_L:2 * _L].T, wih[2 * _L:].T,
        _row(bih[:_L]), _row(bih[_L:2 * _L]), _row(bih[2 * _L:]),
        _row(bhh[:_L]), _row(bhh[_L:2 * _L]), _row(bhh[2 * _L:]),
        d["l1"]["W"].T, _row(d["l1"]["b"]),
        d["l2"]["W"].T, _row(d["l2"]["b"]),
        wd3, bd3)
    return out_pad[:, :3], memory


# speedup vs baseline: 27.9101x; 2.4515x over previous
import functools

import jax
import jax.numpy as jnp
from jax import lax
from jax.experimental import pallas as pl
from jax.experimental.pallas import tpu as pltpu
from jax.experimental.pallas import tpu_sc as plsc

_N = 10000
_E = 320000
_L = 128

_NPAD = 10240
_NW = 32
_ROWS = _NPAD // _NW
_CAPS = 12400
_CAPA = _CAPS + 16
_RCAP = _NW * _CAPS
_ET = _E + _RCAP
_CPW = _ET // _NW // 128

def _mesh():
    return plsc.VectorSubcoreMesh(core_axis_name="c", subcore_axis_name="s")


def _bf16_round(v):
    b = plsc.bitcast(v, jnp.int32)
    r = (b + 0x7FFF + ((b >> 16) & 1)) & (-65536)
    return plsc.bitcast(r, jnp.float32)


def _sc_radius_body(cx_hbm, cy_hbm, src_o, dst_o, dx_o, dy_o, val_o,
                    cx_v, cy_v, cxb_v, cyb_v, sb, db, xb, yb, vb):
    w = lax.axis_index("c") * 16 + lax.axis_index("s")
    pltpu.sync_copy(cx_hbm, cx_v)
    pltpu.sync_copy(cy_hbm, cy_v)
    zi = jnp.zeros((16,), jnp.int32)
    zf = jnp.zeros((16,), jnp.float32)

    def cbody(k, carry):
        off = k * 16
        cxb_v[pl.ds(off, 16)] = _bf16_round(cx_v[pl.ds(off, 16)])
        cyb_v[pl.ds(off, 16)] = _bf16_round(cy_v[pl.ds(off, 16)])
        return carry

    lax.fori_loop(0, _NPAD // 16, cbody, 0)

    def zbody(k, carry):
        off = k * 16
        sb[pl.ds(off, 16)] = zi
        db[pl.ds(off, 16)] = zi
        xb[pl.ds(off, 16)] = zf
        yb[pl.ds(off, 16)] = zf
        vb[pl.ds(off, 16)] = zf
        return carry

    lax.fori_loop(0, _CAPA // 16, zbody, 0)

    iota = lax.iota(jnp.int32, 16)
    ones = jnp.ones((16,), jnp.float32)

    def rbody(r, cnt):
        i = w * _ROWS + r
        isp = jnp.full((16,), i, jnp.int32)
        ib = (i // 16) * 16
        sel = iota == (i - ib)
        cxi_s = jnp.sum(jnp.where(sel, cx_v[pl.ds(ib, 16)], 0.0))
        cyi_s = jnp.sum(jnp.where(sel, cy_v[pl.ds(ib, 16)], 0.0))
        cxbi_s = jnp.sum(jnp.where(sel, cxb_v[pl.ds(ib, 16)], 0.0))
        cybi_s = jnp.sum(jnp.where(sel, cyb_v[pl.ds(ib, 16)], 0.0))
        cxi = jnp.full((16,), cxi_s, jnp.float32)
        cyi = jnp.full((16,), cyi_s, jnp.float32)
        cxbi = jnp.full((16,), cxbi_s, jnp.float32)
        cybi = jnp.full((16,), cybi_s, jnp.float32)
        sqi = cxi * cxi + cyi * cyi

        def jbody(jj, cnt):
            jo = jj * 16
            cxj = cx_v[pl.ds(jo, 16)]
            cyj = cy_v[pl.ds(jo, 16)]
            dot = cxb_v[pl.ds(jo, 16)] * cxbi + cyb_v[pl.ds(jo, 16)] * cybi
            sqj = cxj * cxj + cyj * cyj
            d2 = (sqi + sqj) - 2.0 * dot
            jidx = jo + iota
            msk = (d2 < 0.01) & (jidx != isp)
            c = jnp.minimum(cnt, _CAPS)
            plsc.store_compressed(sb.at[pl.ds(c, 16)], isp, mask=msk)
            plsc.store_compressed(db.at[pl.ds(c, 16)], jidx, mask=msk)
            plsc.store_compressed(xb.at[pl.ds(c, 16)], cxj - cxi, mask=msk)
            plsc.store_compressed(yb.at[pl.ds(c, 16)], cyj - cyi, mask=msk)
            plsc.store_compressed(vb.at[pl.ds(c, 16)], ones, mask=msk)
            return cnt + jnp.sum(msk.astype(jnp.int32))

        return lax.fori_loop(0, _NPAD // 16, jbody, cnt)

    lax.fori_loop(0, _ROWS, rbody, jnp.int32(0))
    pltpu.sync_copy(sb.at[pl.ds(0, _CAPS)], src_o.at[w])
    pltpu.sync_copy(db.at[pl.ds(0, _CAPS)], dst_o.at[w])
    pltpu.sync_copy(xb.at[pl.ds(0, _CAPS)], dx_o.at[w])
    pltpu.sync_copy(yb.at[pl.ds(0, _CAPS)], dy_o.at[w])
    pltpu.sync_copy(vb.at[pl.ds(0, _CAPS)], val_o.at[w])


@functools.cache
def _sc_radius_kernel():
    return pl.kernel(
        _sc_radius_body,
        out_type=[
            jax.ShapeDtypeStruct((_NW, _CAPS), jnp.int32),
            jax.ShapeDtypeStruct((_NW, _CAPS), jnp.int32),
            jax.ShapeDtypeStruct((_NW, _CAPS), jnp.float32),
            jax.ShapeDtypeStruct((_NW, _CAPS), jnp.float32),
            jax.ShapeDtypeStruct((_NW, _CAPS), jnp.float32),
        ],
        mesh=_mesh(),
        compiler_params=pltpu.CompilerParams(needs_layout_passes=False, use_tc_tiling_on_sc=False),
        scratch_types=[
            pltpu.VMEM((_NPAD,), jnp.float32),
            pltpu.VMEM((_NPAD,), jnp.float32),
            pltpu.VMEM((_NPAD,), jnp.float32),
            pltpu.VMEM((_NPAD,), jnp.float32),
            pltpu.VMEM((_CAPA,), jnp.int32),
            pltpu.VMEM((_CAPA,), jnp.int32),
            pltpu.VMEM((_CAPA,), jnp.float32),
            pltpu.VMEM((_CAPA,), jnp.float32),
            pltpu.VMEM((_CAPA,), jnp.float32),
        ],
    )


def _sc_radius(cxp, cyp):
    return _sc_radius_kernel()(cxp, cyp)


_NBUF = 5
_TPW = _CPW // _NBUF


def _sc_gather_body(u_hbm, idx_hbm, out_hbm, idx2d, r0, r1, r2, r3, r4,
                    u_sh, g0, g1, g2, g3, g4, w0, w1, w2, w3, w4):
    c = lax.axis_index("c")
    s = lax.axis_index("s")
    w = c * 16 + s
    base = w * (_CPW * 128)
    rows = [r0, r1, r2, r3, r4]
    gs = [g0, g1, g2, g3, g4]
    ws = [w0, w1, w2, w3, w4]

    r0_ = s * (_N // 16)
    pltpu.sync_copy(u_hbm.at[pl.ds(r0_, _N // 16)], u_sh.at[pl.ds(r0_, _N // 16)])
    plsc.subcore_barrier()

    for b in range(_NBUF):
        pltpu.sync_copy(idx_hbm.at[pl.ds(base + b * 128, 128)], idx2d.at[b])
        pltpu.async_copy(u_sh.at[idx2d.at[b]], rows[b], gs[b])

    def mbody(tb, carry):
        for b in range(_NBUF):
            t = tb * _NBUF + b
            off = base + t * 128
            pltpu.make_async_copy(
                u_sh.at[idx2d.at[b]], rows[b], gs[b]).wait()
            pltpu.async_copy(rows[b], out_hbm.at[pl.ds(off, 128)], ws[b])

            @pl.when(tb < _TPW - 1)
            def _():
                pltpu.make_async_copy(
                    rows[b], out_hbm.at[pl.ds(base, 128)], ws[b]).wait()
                pltpu.sync_copy(
                    idx_hbm.at[pl.ds(base + (t + _NBUF) * 128, 128)], idx2d.at[b])
                pltpu.async_copy(u_sh.at[idx2d.at[b]], rows[b], gs[b])
        return carry

    lax.fori_loop(0, _TPW, mbody, 0)
    for b in range(_NBUF):
        pltpu.make_async_copy(rows[b], out_hbm.at[pl.ds(base, 128)], ws[b]).wait()


@functools.cache
def _sc_gather_kernel():
    return pl.kernel(
        _sc_gather_body,
        out_type=jax.ShapeDtypeStruct((_ET, _L), jnp.bfloat16),
        mesh=_mesh(),
        compiler_params=pltpu.CompilerParams(needs_layout_passes=False, use_tc_tiling_on_sc=False),
        scratch_types=[
            pltpu.VMEM((_NBUF, 128), jnp.int32),
        ]
        + [pltpu.VMEM((128, _L), jnp.bfloat16)] * _NBUF
        + [pltpu.VMEM_SHARED((_N, _L), jnp.bfloat16)]
        + [pltpu.SemaphoreType.DMA] * (2 * _NBUF),
    )


def _sc_gather(u, idx):
    return _sc_gather_kernel()(u.astype(jnp.bfloat16), idx)


def _sc_scatter_body(m_hbm, tgt_hbm, zeros_hbm, out_hbm, idx_v, mb, acc):
    c = lax.axis_index("c")
    s = lax.axis_index("s")
    w = c * 16 + s

    def zbody(q, carry):
        r0 = s * 625 + q * 125
        pltpu.sync_copy(zeros_hbm, acc.at[pl.ds(r0, 125)])
        return carry

    lax.fori_loop(0, 5, zbody, 0)
    plsc.subcore_barrier()

    base = w * (_CPW * 128)

    def body(t, carry):
        off = base + t * 128
        pltpu.sync_copy(tgt_hbm.at[pl.ds(off, 128)], idx_v.at[0])
        pltpu.sync_copy(m_hbm.at[pl.ds(off, 128)], mb)
        pltpu.sync_copy(mb, acc.at[idx_v.at[0]], add=True)
        return carry

    lax.fori_loop(0, _CPW, body, 0)
    plsc.subcore_barrier()

    def obody(q, carry):
        r0 = s * 625 + q * 125
        pltpu.sync_copy(acc.at[pl.ds(r0, 125)], out_hbm.at[c, pl.ds(r0, 125)])
        return carry

    lax.fori_loop(0, 5, obody, 0)


@functools.cache
def _sc_scatter_kernel():
    return pl.kernel(
        _sc_scatter_body,
        out_type=jax.ShapeDtypeStruct((2, _N, _L), jnp.float32),
        mesh=_mesh(),
        compiler_params=pltpu.CompilerParams(needs_layout_passes=False, use_tc_tiling_on_sc=False),
        scratch_types=[
            pltpu.VMEM((2, 128), jnp.int32),
            pltpu.VMEM((128, _L), jnp.float32),
            pltpu.VMEM_SHARED((_N, _L), jnp.float32),
        ],
    )


def _sc_scatter(m, tgt, zeros_chunk):
    return _sc_scatter_kernel()(m, tgt, zeros_chunk)


_BN = 1000
_BE = 1024


def _tc_node(xp, gfp, wn1, bn1, wn2, bn2, wg1, bg1, wg2, bg2):
    def body(x_ref, gf_ref, wn1r, bn1r, wn2r, bn2r, wg1r, bg1r, wg2r, bg2r, o_ref):
        h = jnp.maximum(x_ref[...] @ wn1r[...] + bn1r[...], 0.0)
        h = jnp.maximum(h @ wn2r[...] + bn2r[...], 0.0)
        gg = jnp.maximum(gf_ref[...] @ wg1r[...] + bg1r[...], 0.0)
        gg = gg @ wg2r[...] + bg2r[...]
        o_ref[...] = h + gg[0:1, :]

    full = lambda shape: pl.BlockSpec(shape, lambda i: (0, 0))
    return pl.pallas_call(
        body,
        grid=(_N // _BN,),
        in_specs=[
            pl.BlockSpec((_BN, 128), lambda i: (i, 0)),
            full((8, 128)), full((128, 128)), full((1, 128)),
            full((128, 128)), full((1, 128)),
            full((128, 128)), full((1, 128)),
            full((128, 128)), full((1, 128)),
        ],
        out_specs=pl.BlockSpec((_BN, _L), lambda i: (i, 0)),
        out_shape=jax.ShapeDtypeStruct((_N, _L), jnp.float32),
    )(xp, gfp, wn1, bn1, wn2, bn2, wg1, bg1, wg2, bg2)


def _tc_edge_enc(feat, w1, b1, w2, b2):
    def body(f_ref, w1r, b1r, w2r, b2r, o_ref):
        h = jnp.maximum(f_ref[...] @ w1r[...] + b1r[...], 0.0)
        o_ref[...] = jnp.maximum(h @ w2r[...] + b2r[...], 0.0)

    full = lambda shape: pl.BlockSpec(shape, lambda i: (0, 0))
    return pl.pallas_call(
        body,
        grid=(_ET // _BE,),
        in_specs=[
            pl.BlockSpec((_BE, 8), lambda i: (i, 0)),
            full((8, 128)), full((1, 128)),
            full((128, 128)), full((1, 128)),
        ],
        out_specs=pl.BlockSpec((_BE, _L), lambda i: (i, 0)),
        out_shape=jax.ShapeDtypeStruct((_ET, _L), jnp.float32),
    )(feat, w1, b1, w2, b2)


def _tc_lin(x, w, b):
    def body(x_ref, w_ref, b_ref, o_ref):
        o_ref[...] = x_ref[...] @ w_ref[...] + b_ref[...]

    full = lambda shape: pl.BlockSpec(shape, lambda i: (0, 0))
    return pl.pallas_call(
        body,
        grid=(_N // _BN,),
        in_specs=[
            pl.BlockSpec((_BN, 128), lambda i: (i, 0)),
            full((128, 128)), full((1, 128)),
        ],
        out_specs=pl.BlockSpec((_BN, 128), lambda i: (i, 0)),
        out_shape=jax.ShapeDtypeStruct((_N, 128), jnp.float32),
    )(x, w, b)


def _tc_msg(g, e, valid, w1b, w2, b2):
    def body(g_ref, e_ref, v_ref, w1r, w2r, b2r, o_ref):
        m1 = jnp.maximum(g_ref[...].astype(jnp.float32) + e_ref[...] @ w1r[...], 0.0)
        m = jnp.maximum(m1 @ w2r[...] + b2r[...], 0.0)
        o_ref[...] = jnp.where(v_ref[...] > 0.0, m, 0.0)

    full = lambda shape: pl.BlockSpec(shape, lambda i: (0, 0))
    return pl.pallas_call(
        body,
        grid=(_ET // _BE,),
        in_specs=[
            pl.BlockSpec((_BE, _L), lambda i: (i, 0)),
            pl.BlockSpec((_BE, _L), lambda i: (i, 0)),
            pl.BlockSpec((_BE, 1), lambda i: (i, 0)),
            full((128, 128)), full((128, 128)), full((1, 128)),
        ],
        out_specs=pl.BlockSpec((_BE, _L), lambda i: (i, 0)),
        out_shape=jax.ShapeDtypeStruct((_ET, _L), jnp.float32),
    )(g, e, valid, w1b, w2, b2)


def _tc_upd(x_enc, p0, p1, wua, wub, b1, w2, b2):
    def body(x_ref, p0_ref, p1_ref, wuar, wubr, b1r, w2r, b2r, o_ref):
        mv = p0_ref[...] + p1_ref[...]
        t = jnp.maximum(x_ref[...] @ wuar[...] + mv @ wubr[...] + b1r[...], 0.0)
        t = jnp.maximum(t @ w2r[...] + b2r[...], 0.0)
        o_ref[...] = jnp.maximum(x_ref[...] + t, 0.0)

    full = lambda shape: pl.BlockSpec(shape, lambda i: (0, 0))
    return pl.pallas_call(
        body,
        grid=(_N // _BN,),
        in_specs=[
            pl.BlockSpec((_BN, _L), lambda i: (i, 0)),
            pl.BlockSpec((_BN, _L), lambda i: (i, 0)),
            pl.BlockSpec((_BN, _L), lambda i: (i, 0)),
            full((128, 128)), full((128, 128)), full((1, 128)),
            full((128, 128)), full((1, 128)),
        ],
        out_specs=pl.BlockSpec((_BN, _L), lambda i: (i, 0)),
        out_shape=jax.ShapeDtypeStruct((_N, _L), jnp.float32),
    )(x_enc, p0, p1, wua, wub, b1, w2, b2)


def _tc_head(x_enc, wr, wz, wn, br, bz, bn, hr, hz, hn, wd1, bd1, wd2, bd2, wd3, bd3):
    def body(x_ref, wrr, wzr, wnr, brr, bzr, bnr, hrr, hzr, hnr,
             wd1r, bd1r, wd2r, bd2r, wd3r, bd3r, o_ref, mem_ref):
        x = x_ref[...]
        r = jax.nn.sigmoid(x @ wrr[...] + brr[...] + hrr[...])
        z = jax.nn.sigmoid(x @ wzr[...] + bzr[...] + hzr[...])
        nn_ = jnp.tanh(x @ wnr[...] + bnr[...] + r * hnr[...])
        mem = (1.0 - z) * nn_
        h = jnp.maximum(mem @ wd1r[...] + bd1r[...], 0.0)
        h = jnp.maximum(h @ wd2r[...] + bd2r[...], 0.0)
        o_ref[...] = h @ wd3r[...] + bd3r[...]
        mem_ref[...] = mem

    full = lambda shape: pl.BlockSpec(shape, lambda i: (0, 0))
    return pl.pallas_call(
        body,
        grid=(_N // _BN,),
        in_specs=[pl.BlockSpec((_BN, _L), lambda i: (i, 0))]
        + [full((128, 128)), full((128, 128)), full((128, 128))]
        + [full((1, 128))] * 6
        + [full((128, 128)), full((1, 128))] * 3,
        out_specs=[
            pl.BlockSpec((_BN, 128), lambda i: (i, 0)),
            pl.BlockSpec((_BN, _L), lambda i: (i, 0)),
        ],
        out_shape=[
            jax.ShapeDtypeStruct((_N, 128), jnp.float32),
            jax.ShapeDtypeStruct((_N, _L), jnp.float32),
        ],
    )(x_enc, wr, wz, wn, br, bz, bn, hr, hz, hn, wd1, bd1, wd2, bd2, wd3, bd3)


def _row(b):
    return b.reshape(1, -1)


def kernel(x, edge_index, edge_features, global_features, params):
    f32 = jnp.float32
    coords = x[:, :2]
    pad = 100.0 + 4.0 * jnp.arange(_NPAD - _N, dtype=f32)
    cxp = jnp.concatenate([coords[:, 0], pad])
    cyp = jnp.concatenate([coords[:, 1], jnp.zeros((_NPAD - _N,), f32)])

    src32, dst32, dx32, dy32, val32 = _sc_radius(cxp, cyp)
    src_r = src32.reshape(-1)
    dst_r = dst32.reshape(-1)

    src_all = jnp.concatenate([edge_index[0].astype(jnp.int32), src_r])
    tgt_all = jnp.concatenate([edge_index[1].astype(jnp.int32), dst_r])
    valid = jnp.concatenate([jnp.ones((_E, 1), f32), val32.reshape(-1, 1)])

    feat = jnp.concatenate([
        jnp.pad(edge_features, ((0, 0), (0, 6))),
        jnp.concatenate([dx32.reshape(-1, 1), dy32.reshape(-1, 1),
                         jnp.zeros((_RCAP, 6), f32)], axis=1),
    ], axis=0)

    p = params
    xp = jnp.pad(x, ((0, 0), (0, 125)))
    gfp = jnp.pad(global_features.reshape(1, 3), ((0, 7), (0, 125)))

    wn1 = jnp.pad(p["node"]["l1"]["W"].T, ((0, 125), (0, 0)))
    wg1 = jnp.pad(p["glob"]["l1"]["W"].T, ((0, 125), (0, 0)))
    x_enc = _tc_node(xp, gfp,
                     wn1, _row(p["node"]["l1"]["b"]),
                     p["node"]["l2"]["W"].T, _row(p["node"]["l2"]["b"]),
                     wg1, _row(p["glob"]["l1"]["b"]),
                     p["glob"]["l2"]["W"].T, _row(p["glob"]["l2"]["b"]))

    we1 = jnp.pad(p["edge"]["l1"]["W"].T, ((0, 6), (0, 0)))
    e_all = _tc_edge_enc(feat, we1, _row(p["edge"]["l1"]["b"]),
                         p["edge"]["l2"]["W"].T, _row(p["edge"]["l2"]["b"]))

    zeros_chunk = jnp.zeros((125, _L), f32)
    for pm, pu in zip(p["msg"], p["upd"]):
        w1 = pm["l1"]["W"]
        u = _tc_lin(x_enc, w1[:, :_L].T, _row(pm["l1"]["b"]))
        gth = _sc_gather(u, src_all)
        m = _tc_msg(gth, e_all, valid, w1[:, _L:].T,
                    pm["l2"]["W"].T, _row(pm["l2"]["b"]))
        parts = _sc_scatter(m, tgt_all, zeros_chunk)
        wu = pu["l1"]["W"]
        x_enc = _tc_upd(x_enc, parts[0], parts[1],
                        wu[:, :_L].T, wu[:, _L:].T, _row(pu["l1"]["b"]),
                        pu["l2"]["W"].T, _row(pu["l2"]["b"]))

    gru = p["gru"]
    wih = gru["Wih"]
    bih = gru["bih"]
    bhh = gru["bhh"]
    d = p["dec"]
    wd3 = jnp.pad(d["l3"]["W"].T, ((0, 0), (0, 125)))
    bd3 = jnp.pad(_row(d["l3"]["b"]), ((0, 0), (0, 125)))
    out_pad, memory = _tc_head(
        x_enc,
        wih[:_L].T, wih[_L:2 * _L].T, wih[2 * _L:].T,
        _row(bih[:_L]), _row(bih[_L:2 * _L]), _row(bih[2 * _L:]),
        _row(bhh[:_L]), _row(bhh[_L:2 * _L]), _row(bhh[2 * _L:]),
        d["l1"]["W"].T, _row(d["l1"]["b"]),
        d["l2"]["W"].T, _row(d["l2"]["b"]),
        wd3, bd3)
    return out_pad[:, :3], memory
